# Initial kernel scaffold; baseline (speedup 1.0000x reference)
import functools

import jax
import jax.numpy as jnp
from jax import lax
from jax.experimental import pallas as pl
from jax.experimental.pallas import tpu as pltpu, tpu_sc as plsc

N = 10000
E = 320000
D = 128
H = 128
C = 2

NC = 2
NS = 16
LANES = 128
CPW = 79
EPW = CPW * LANES
E_PAD = NC * NS * EPW
ROWS_PER_TILE = 640
N_PAD = NS * ROWS_PER_TILE

_mesh = plsc.VectorSubcoreMesh(core_axis_name="c", subcore_axis_name="s")


def _fill2d(ref, nrows, ncols, val):
    def row(i, _):
        def col(j, __):
            ref[i, pl.ds(j * 16, 16)] = jnp.full((16,), val, jnp.float32)
            return 0
        return lax.fori_loop(0, ncols // 16, col, 0)
    lax.fori_loop(0, nrows, row, 0)


@functools.partial(
    pl.kernel,
    out_type=jax.ShapeDtypeStruct((NC, N_PAD, 16), jnp.float32),
    mesh=_mesh,
    scratch_types=[
        pltpu.VMEM((CPW, LANES), jnp.int32),
        pltpu.VMEM((LANES, 16), jnp.float32),
        pltpu.VMEM((LANES, 16), jnp.float32),
        pltpu.VMEM_SHARED((N_PAD, 16), jnp.float32),
    ],
)
def _deg_kernel(dst_hbm, out_hbm, dst_v, ones_v, z_v, acc):
    c = lax.axis_index("c")
    s = lax.axis_index("s")
    pltpu.sync_copy(dst_hbm.at[c, s], dst_v)
    _fill2d(ones_v, LANES, 16, 1.0)
    _fill2d(z_v, LANES, 16, 0.0)
    for k in range(ROWS_PER_TILE // LANES):
        pltpu.sync_copy(z_v, acc.at[pl.ds(s * ROWS_PER_TILE + k * LANES, LANES)])
    plsc.subcore_barrier()

    def chunk(j, _):
        pltpu.sync_copy(ones_v, acc.at[dst_v.at[j]], add=True)
        return 0
    lax.fori_loop(0, CPW, chunk, 0)
    plsc.subcore_barrier()
    pltpu.sync_copy(acc.at[pl.ds(s * ROWS_PER_TILE, ROWS_PER_TILE)],
                    out_hbm.at[c, pl.ds(s * ROWS_PER_TILE, ROWS_PER_TILE)])


@functools.partial(
    pl.kernel,
    out_type=jax.ShapeDtypeStruct((NC, N_PAD, D), jnp.float32),
    mesh=_mesh,
    scratch_types=[
        pltpu.VMEM((CPW, LANES), jnp.int32),
        pltpu.VMEM((CPW, LANES), jnp.int32),
        pltpu.VMEM((LANES, D), jnp.float32),
        pltpu.VMEM((LANES, D), jnp.float32),
        pltpu.SemaphoreType.DMA,
        pltpu.VMEM_SHARED((N_PAD, D), jnp.float32),
    ],
)
def _agg_kernel(u_hbm, src_hbm, dst_hbm, out_hbm, src_v, dst_v, rows_v, z_v, sem, acc):
    c = lax.axis_index("c")
    s = lax.axis_index("s")
    pltpu.sync_copy(src_hbm.at[c, s], src_v)
    pltpu.sync_copy(dst_hbm.at[c, s], dst_v)
    _fill2d(z_v, LANES, D, 0.0)
    for k in range(ROWS_PER_TILE // LANES):
        pltpu.sync_copy(z_v, acc.at[pl.ds(s * ROWS_PER_TILE + k * LANES, LANES)])
    plsc.subcore_barrier()

    def chunk(j, _):
        pltpu.async_copy(u_hbm.at[src_v.at[j]], rows_v, sem).wait()
        pltpu.sync_copy(rows_v, acc.at[dst_v.at[j]], add=True)
        return 0
    lax.fori_loop(0, CPW, chunk, 0)
    plsc.subcore_barrier()
    pltpu.sync_copy(acc.at[pl.ds(s * ROWS_PER_TILE, ROWS_PER_TILE)],
                    out_hbm.at[c, pl.ds(s * ROWS_PER_TILE, ROWS_PER_TILE)])


BN = 1000
_GRID = N // BN


def _row_spec():
    return pl.BlockSpec((BN, 128), lambda i: (i, 0))


def _tc1_body(x_ref, w_ref, d0_ref, d1_ref, u_ref, dinv_ref):
    deg = 1.0 + d0_ref[:, 0:1] + d1_ref[:, 0:1]
    dinv = lax.rsqrt(deg)
    dinv_b = jnp.broadcast_to(dinv, (BN, 128))
    p = jnp.dot(x_ref[...], w_ref[...], preferred_element_type=jnp.float32)
    u_ref[...] = dinv_b * p
    dinv_ref[...] = dinv_b


def _tc1(x, w1, d0, d1):
    return pl.pallas_call(
        _tc1_body,
        grid=(_GRID,),
        in_specs=[
            _row_spec(),
            pl.BlockSpec((128, 128), lambda i: (0, 0)),
            pl.BlockSpec((BN, 16), lambda i: (i, 0)),
            pl.BlockSpec((BN, 16), lambda i: (i, 0)),
        ],
        out_specs=[_row_spec(), _row_spec()],
        out_shape=[
            jax.ShapeDtypeStruct((N, 128), jnp.float32),
            jax.ShapeDtypeStruct((N, 128), jnp.float32),
        ],
    )(x, w1, d0, d1)


def _tc_mid_body(a0_ref, a1_ref, u_ref, dinv_ref, b_ref, w_ref, out_ref):
    h = dinv_ref[...] * (a0_ref[...] + a1_ref[...] + u_ref[...]) + b_ref[...]
    h = jnp.maximum(h, 0.0)
    out_ref[...] = dinv_ref[...] * jnp.dot(
        h, w_ref[...], preferred_element_type=jnp.float32)


def _tc_mid(a0, a1, u, dinv_b, b, w):
    return pl.pallas_call(
        _tc_mid_body,
        grid=(_GRID,),
        in_specs=[
            _row_spec(), _row_spec(), _row_spec(), _row_spec(),
            pl.BlockSpec((1, 128), lambda i: (0, 0)),
            pl.BlockSpec((128, 128), lambda i: (0, 0)),
        ],
        out_specs=_row_spec(),
        out_shape=jax.ShapeDtypeStruct((N, 128), jnp.float32),
    )(a0, a1, u, dinv_b, b, w)


def _tc3_body(a0_ref, a1_ref, u_ref, dinv_ref, b_ref, out_ref):
    h = dinv_ref[...] * (a0_ref[...] + a1_ref[...] + u_ref[...]) + b_ref[...]
    out_ref[...] = dinv_ref[...] * jnp.maximum(h, 0.0)


def _tc3(a0, a1, u, dinv_b, b):
    return pl.pallas_call(
        _tc3_body,
        grid=(_GRID,),
        in_specs=[
            _row_spec(), _row_spec(), _row_spec(), _row_spec(),
            pl.BlockSpec((1, 128), lambda i: (0, 0)),
        ],
        out_specs=_row_spec(),
        out_shape=jax.ShapeDtypeStruct((N, 128), jnp.float32),
    )(a0, a1, u, dinv_b, b)


def _tc4_body(a0_ref, a1_ref, v_ref, dinv_ref, w_ref, b_ref, out_ref):
    z = dinv_ref[...] * (a0_ref[...] + a1_ref[...] + v_ref[...])
    out_ref[...] = jnp.dot(z, w_ref[...],
                           preferred_element_type=jnp.float32) + b_ref[...]


def _tc4(a0, a1, v, dinv_b, w3p, b3p):
    return pl.pallas_call(
        _tc4_body,
        grid=(_GRID,),
        in_specs=[
            _row_spec(), _row_spec(), _row_spec(), _row_spec(),
            pl.BlockSpec((128, 128), lambda i: (0, 0)),
            pl.BlockSpec((1, 128), lambda i: (0, 0)),
        ],
        out_specs=_row_spec(),
        out_shape=jax.ShapeDtypeStruct((N, 128), jnp.float32),
    )(a0, a1, v, dinv_b, w3p, b3p)


def kernel(x, edge_index, W1, b1, W2, b2, W3, b3):
    src = edge_index[0]
    dst = edge_index[1]
    pad = E_PAD - E
    src_p = jnp.concatenate([src, jnp.zeros((pad,), jnp.int32)])
    dst_p = jnp.concatenate([dst, jnp.full((pad,), N, jnp.int32)])
    src3 = src_p.reshape(NC, NS, CPW, LANES)
    dst3 = dst_p.reshape(NC, NS, CPW, LANES)

    degp = _deg_kernel(dst3)
    d0 = degp[0, :N]
    d1 = degp[1, :N]

    u1, dinv_b = _tc1(x, W1, d0, d1)
    a1 = _agg_kernel(u1, src3, dst3)
    u2 = _tc_mid(a1[0, :N], a1[1, :N], u1, dinv_b, b1.reshape(1, 128), W2)
    a2 = _agg_kernel(u2, src3, dst3)
    v3 = _tc3(a2[0, :N], a2[1, :N], u2, dinv_b, b2.reshape(1, 128))
    a3 = _agg_kernel(v3, src3, dst3)
    w3p = jnp.pad(W3, ((0, 0), (0, 128 - C)))
    b3p = jnp.pad(b3, (0, 128 - C)).reshape(1, 128)
    outp = _tc4(a3[0, :N], a3[1, :N], v3, dinv_b, w3p, b3p)
    return outp[:, :C]


# speedup vs baseline: 10.5100x; 10.5100x over previous
import functools

import jax
import jax.numpy as jnp
from jax import lax
from jax.experimental import pallas as pl
from jax.experimental.pallas import tpu as pltpu, tpu_sc as plsc

N = 10000
E = 320000
D = 128
H = 128
C = 2

NC = 2
NS = 16
LANES = 128
CPW = 79
EPW = CPW * LANES
E_PAD = NC * NS * EPW
ROWS_PER_TILE = 640
N_PAD = NS * ROWS_PER_TILE

_mesh = plsc.VectorSubcoreMesh(core_axis_name="c", subcore_axis_name="s")


def _fill2d(ref, nrows, ncols, val):
    def row(i, _):
        def col(j, __):
            ref[i, pl.ds(j * 16, 16)] = jnp.full((16,), val, jnp.float32)
            return 0
        return lax.fori_loop(0, ncols // 16, col, 0)
    lax.fori_loop(0, nrows, row, 0)


@functools.partial(
    pl.kernel,
    out_type=jax.ShapeDtypeStruct((NC, N_PAD, 16), jnp.float32),
    mesh=_mesh,
    scratch_types=[
        pltpu.VMEM((CPW, LANES), jnp.int32),
        pltpu.VMEM((LANES, 16), jnp.float32),
        pltpu.VMEM((LANES, 16), jnp.float32),
        pltpu.VMEM_SHARED((N_PAD, 16), jnp.float32),
    ],
)
def _deg_kernel(dst_hbm, out_hbm, dst_v, ones_v, z_v, acc):
    c = lax.axis_index("c")
    s = lax.axis_index("s")
    pltpu.sync_copy(dst_hbm.at[c, s], dst_v)
    _fill2d(ones_v, LANES, 16, 1.0)
    _fill2d(z_v, LANES, 16, 0.0)
    for k in range(ROWS_PER_TILE // LANES):
        pltpu.sync_copy(z_v, acc.at[pl.ds(s * ROWS_PER_TILE + k * LANES, LANES)])
    plsc.subcore_barrier()

    def chunk(j, _):
        pltpu.sync_copy(ones_v, acc.at[dst_v.at[j]], add=True)
        return 0
    lax.fori_loop(0, CPW, chunk, 0)
    plsc.subcore_barrier()
    pltpu.sync_copy(acc.at[pl.ds(s * ROWS_PER_TILE, ROWS_PER_TILE)],
                    out_hbm.at[c, pl.ds(s * ROWS_PER_TILE, ROWS_PER_TILE)])


@functools.partial(
    pl.kernel,
    out_type=jax.ShapeDtypeStruct((NC, N_PAD, D), jnp.float32),
    mesh=_mesh,
    scratch_types=[
        pltpu.VMEM((CPW, LANES), jnp.int32),
        pltpu.VMEM((CPW, LANES), jnp.int32),
        pltpu.VMEM((LANES, D), jnp.float32),
        pltpu.SemaphoreType.DMA,
        pltpu.VMEM_SHARED((N_PAD, D), jnp.float32),
    ],
)
def _agg_kernel(u_hbm, src_hbm, dst_hbm, out_hbm, src_v, dst_v, rows_v, sem, acc):
    c = lax.axis_index("c")
    s = lax.axis_index("s")
    pltpu.sync_copy(src_hbm.at[c, s], src_v)
    pltpu.sync_copy(dst_hbm.at[c, s], dst_v)
    _fill2d(rows_v, LANES, D, 0.0)
    for k in range(ROWS_PER_TILE // LANES):
        pltpu.sync_copy(rows_v, acc.at[pl.ds(s * ROWS_PER_TILE + k * LANES, LANES)])
    plsc.subcore_barrier()

    def chunk(j, _):
        pltpu.async_copy(u_hbm.at[src_v.at[j]], rows_v, sem).wait()
        pltpu.sync_copy(rows_v, acc.at[dst_v.at[j]], add=True)
        return 0
    lax.fori_loop(0, CPW, chunk, 0)
    plsc.subcore_barrier()
    pltpu.sync_copy(acc.at[pl.ds(s * ROWS_PER_TILE, ROWS_PER_TILE)],
                    out_hbm.at[c, pl.ds(s * ROWS_PER_TILE, ROWS_PER_TILE)])


BN = 1000
_GRID = N // BN


def _row_spec():
    return pl.BlockSpec((BN, 128), lambda i: (i, 0))


def _tc1_body(x_ref, w_ref, d0_ref, d1_ref, u_ref, dinv_ref):
    deg = 1.0 + d0_ref[:, 0:1] + d1_ref[:, 0:1]
    dinv = lax.rsqrt(deg)
    dinv_b = jnp.broadcast_to(dinv, (BN, 128))
    p = jnp.dot(x_ref[...], w_ref[...], preferred_element_type=jnp.float32)
    u_ref[...] = dinv_b * p
    dinv_ref[...] = dinv_b


def _tc1(x, w1, d0, d1):
    return pl.pallas_call(
        _tc1_body,
        grid=(_GRID,),
        in_specs=[
            _row_spec(),
            pl.BlockSpec((128, 128), lambda i: (0, 0)),
            pl.BlockSpec((BN, 16), lambda i: (i, 0)),
            pl.BlockSpec((BN, 16), lambda i: (i, 0)),
        ],
        out_specs=[_row_spec(), _row_spec()],
        out_shape=[
            jax.ShapeDtypeStruct((N, 128), jnp.float32),
            jax.ShapeDtypeStruct((N, 128), jnp.float32),
        ],
    )(x, w1, d0, d1)


def _tc_mid_body(a0_ref, a1_ref, u_ref, dinv_ref, b_ref, w_ref, out_ref):
    h = dinv_ref[...] * (a0_ref[...] + a1_ref[...] + u_ref[...]) + b_ref[...]
    h = jnp.maximum(h, 0.0)
    out_ref[...] = dinv_ref[...] * jnp.dot(
        h, w_ref[...], preferred_element_type=jnp.float32)


def _tc_mid(a0, a1, u, dinv_b, b, w):
    return pl.pallas_call(
        _tc_mid_body,
        grid=(_GRID,),
        in_specs=[
            _row_spec(), _row_spec(), _row_spec(), _row_spec(),
            pl.BlockSpec((1, 128), lambda i: (0, 0)),
            pl.BlockSpec((128, 128), lambda i: (0, 0)),
        ],
        out_specs=_row_spec(),
        out_shape=jax.ShapeDtypeStruct((N, 128), jnp.float32),
    )(a0, a1, u, dinv_b, b, w)


def _tc3_body(a0_ref, a1_ref, u_ref, dinv_ref, b_ref, out_ref):
    h = dinv_ref[...] * (a0_ref[...] + a1_ref[...] + u_ref[...]) + b_ref[...]
    out_ref[...] = dinv_ref[...] * jnp.maximum(h, 0.0)


def _tc3(a0, a1, u, dinv_b, b):
    return pl.pallas_call(
        _tc3_body,
        grid=(_GRID,),
        in_specs=[
            _row_spec(), _row_spec(), _row_spec(), _row_spec(),
            pl.BlockSpec((1, 128), lambda i: (0, 0)),
        ],
        out_specs=_row_spec(),
        out_shape=jax.ShapeDtypeStruct((N, 128), jnp.float32),
    )(a0, a1, u, dinv_b, b)


def _tc4_body(a0_ref, a1_ref, v_ref, dinv_ref, w_ref, b_ref, out_ref):
    z = dinv_ref[...] * (a0_ref[...] + a1_ref[...] + v_ref[...])
    out_ref[...] = jnp.dot(z, w_ref[...],
                           preferred_element_type=jnp.float32) + b_ref[...]


def _tc4(a0, a1, v, dinv_b, w3p, b3p):
    return pl.pallas_call(
        _tc4_body,
        grid=(_GRID,),
        in_specs=[
            _row_spec(), _row_spec(), _row_spec(), _row_spec(),
            pl.BlockSpec((128, 128), lambda i: (0, 0)),
            pl.BlockSpec((1, 128), lambda i: (0, 0)),
        ],
        out_specs=_row_spec(),
        out_shape=jax.ShapeDtypeStruct((N, 128), jnp.float32),
    )(a0, a1, v, dinv_b, w3p, b3p)


def kernel(x, edge_index, W1, b1, W2, b2, W3, b3):
    src = edge_index[0]
    dst = edge_index[1]
    pad = E_PAD - E
    src_p = jnp.concatenate([src, jnp.zeros((pad,), jnp.int32)])
    dst_p = jnp.concatenate([dst, jnp.full((pad,), N, jnp.int32)])
    src3 = src_p.reshape(NC, NS, CPW, LANES)
    dst3 = dst_p.reshape(NC, NS, CPW, LANES)

    degp = _deg_kernel(dst3)
    d0 = degp[0, :N]
    d1 = degp[1, :N]

    u1, dinv_b = _tc1(x, W1, d0, d1)
    a1 = _agg_kernel(u1, src3, dst3)
    u2 = _tc_mid(a1[0, :N], a1[1, :N], u1, dinv_b, b1.reshape(1, 128), W2)
    a2 = _agg_kernel(u2, src3, dst3)
    v3 = _tc3(a2[0, :N], a2[1, :N], u2, dinv_b, b2.reshape(1, 128))
    a3 = _agg_kernel(v3, src3, dst3)
    w3p = jnp.pad(W3, ((0, 0), (0, 128 - C)))
    b3p = jnp.pad(b3, (0, 128 - C)).reshape(1, 128)
    outp = _tc4(a3[0, :N], a3[1, :N], v3, dinv_b, w3p, b3p)
    return outp[:, :C]
